# baseline (speedup 1.0000x reference)
import jax
import jax.numpy as jnp
from jax.experimental import pallas as pl
from jax.experimental.pallas import tpu as pltpu

N = 10000; E = 320000; IN = 9; H = 128; OUT = 128; ED = 3; L = 4; T = 2; B = 512


def _seg_softmax(a, idx, n):
    m = jax.ops.segment_max(a, idx, num_segments=n)
    m = jnp.where(jnp.isfinite(m), m, 0.0)
    ex = jnp.exp(a - m[idx])
    s = jax.ops.segment_sum(ex, idx, num_segments=n)
    return ex / (s[idx] + 1e-16)


def _gru(x, h, Wih, Whh, bih, bhh):
    gi = x @ Wih.T + bih
    gh = h @ Whh.T + bhh
    ir, iz, inn = jnp.split(gi, 3, axis=-1)
    hr, hz, hn = jnp.split(gh, 3, axis=-1)
    r = jax.nn.sigmoid(ir + hr)
    z = jax.nn.sigmoid(iz + hz)
    nn_ = jnp.tanh(inn + r * hn)
    return (1.0 - z) * nn_ + z * h


def _gat(x_src, x_dst, src, dst, W, a_s, a_d, b, n_dst):
    xs = x_src @ W.T
    xd = x_dst @ W.T
    al = (xs * a_s).sum(-1)[src] + (xd * a_d).sum(-1)[dst]
    al = jax.nn.leaky_relu(al, 0.01)
    al = _seg_softmax(al, dst, n_dst)
    out = jax.ops.segment_sum(xs[src] * al[:, None], dst, num_segments=n_dst)
    return out + b


def _final_linear_body(h_ref, w_ref, b_ref, o_ref):
    o_ref[...] = h_ref[...] @ w_ref[...].T + b_ref[...][None, :]


def _final_linear(h, W, b):
    return pl.pallas_call(
        _final_linear_body,
        out_shape=jax.ShapeDtypeStruct((B, OUT), jnp.float32),
    )(h, W, b)


def kernel(x, edge_attr, lin1_W, lin1_b, gate_att_l, gate_att_r, gate_W1, gate_W2, gate_bias, gru0_Wih, gru0_Whh, gru0_bih, gru0_bhh, atom_W, atom_att_src, atom_att_dst, atom_bias, agru_Wih, agru_Whh, agru_bih, agru_bhh, mol_W, mol_att_src, mol_att_dst, mol_bias, mgru_Wih, mgru_Whh, mgru_bih, mgru_bhh, lin2_W, lin2_b, edge_index, batch):
    src, dst = edge_index[0], edge_index[1]
    h0 = jax.nn.leaky_relu(x @ lin1_W.T + lin1_b, 0.01)
    xj = jax.nn.leaky_relu(jnp.concatenate([h0[src], edge_attr], axis=-1) @ gate_W1.T, 0.01)
    al = xj @ gate_att_l + (h0 @ gate_att_r)[dst]
    al = jax.nn.leaky_relu(al, 0.01)
    al = _seg_softmax(al, dst, N)
    h = jax.ops.segment_sum((h0 @ gate_W2.T)[src] * al[:, None], dst, num_segments=N) + gate_bias
    h = jax.nn.elu(h)
    xh = jax.nn.relu(_gru(h, h0, gru0_Wih, gru0_Whh, gru0_bih, gru0_bhh))
    for l in range(L - 1):
        h = jax.nn.elu(_gat(xh, xh, src, dst, atom_W[l], atom_att_src[l], atom_att_dst[l], atom_bias[l], N))
        xh = jax.nn.relu(_gru(h, xh, agru_Wih[l], agru_Whh[l], agru_bih[l], agru_bhh[l]))
    out = jax.nn.relu(jax.ops.segment_sum(xh, batch, num_segments=B))
    row = jnp.arange(N)
    for _ in range(T):
        h = jax.nn.elu(_gat(xh, out, row, batch, mol_W, mol_att_src, mol_att_dst, mol_bias, B))
        out = jax.nn.relu(_gru(h, out, mgru_Wih, mgru_Whh, mgru_bih, mgru_bhh))
    return _final_linear(out, lin2_W, lin2_b)


# speedup vs baseline: 13.6526x; 13.6526x over previous
import functools

import jax
import jax.numpy as jnp
from jax import lax
from jax.experimental import pallas as pl
from jax.experimental.pallas import tpu as pltpu
from jax.experimental.pallas import tpu_sc as plsc

N = 10000
E = 320000
IN = 9
H = 128
OUT = 128
ED = 3
L = 4
T = 2
B = 512

NP = 10240
NC = 2
NS = 16
NW = NC * NS
EPW = E // NW
RPW = NP // NS

_MESH = plsc.VectorSubcoreMesh(core_axis_name="c", subcore_axis_name="s")

F32 = jnp.float32
I32 = jnp.int32


def _wid():
    return lax.axis_index("c") * NS + lax.axis_index("s")


def _zero_vec_ref(ref, n):
    z = jnp.zeros((16,), F32)

    def body(i, _):
        ref[pl.ds(i * 16, 16)] = z
        return 0

    lax.fori_loop(0, n // 16, body, 0)


GCH = 80
GNC = EPW // GCH


def _sc_gate_gather_body(a_hbm, rdst_hbm, src_hbm, dst_hbm,
                         asrc_out, rdstg_out,
                         srcv, dstv, rows, rdbuf, rdst_v, sem):
    wid = _wid()
    base = wid * EPW
    pltpu.sync_copy(rdst_hbm, rdst_v)

    def chunk(c, _):
        off = base + c * GCH
        pltpu.sync_copy(src_hbm.at[pl.ds(off, GCH)], srcv)
        pltpu.sync_copy(dst_hbm.at[pl.ds(off, GCH)], dstv)
        pltpu.async_copy(a_hbm.at[srcv], rows, sem).wait()
        pltpu.sync_copy(rows, asrc_out.at[pl.ds(off, GCH)])

        def grp(g, _):
            d16 = dstv[pl.ds(g * 16, 16)]
            rdbuf[pl.ds(g * 16, 16)] = plsc.load_gather(rdst_v, [d16])
            return 0

        lax.fori_loop(0, GCH // 16, grp, 0)
        pltpu.sync_copy(rdbuf, rdstg_out.at[pl.ds(off, GCH)])
        return 0

    lax.fori_loop(0, GNC, chunk, 0)


def _sc_gate_gather(a, rdst, src, dst):
    return pl.kernel(
        _sc_gate_gather_body,
        out_type=[jax.ShapeDtypeStruct((E, H), F32),
                  jax.ShapeDtypeStruct((E,), F32)],
        mesh=_MESH,
        compiler_params=pltpu.CompilerParams(needs_layout_passes=False),
        scratch_types=[pltpu.VMEM((GCH,), I32),
                       pltpu.VMEM((GCH,), I32),
                       pltpu.VMEM((GCH, H), F32),
                       pltpu.VMEM((GCH,), F32),
                       pltpu.VMEM((NP,), F32),
                       pltpu.SemaphoreType.DMA],
    )(a, rdst, src, dst)


SCH = 2000


def _sc_stats_gate_body(al_hbm, dst_hbm, suma_out, cnt_out,
                        albuf, dstbuf, suma, cnt):
    wid = _wid()
    base = wid * EPW
    _zero_vec_ref(suma, NP)
    _zero_vec_ref(cnt, NP)
    ones = jnp.full((16,), 1.0, F32)

    def chunk(c, _):
        off = base + c * SCH
        pltpu.sync_copy(al_hbm.at[pl.ds(off, SCH)], albuf)
        pltpu.sync_copy(dst_hbm.at[pl.ds(off, SCH)], dstbuf)

        def grp(g, _):
            d16 = dstbuf[pl.ds(g * 16, 16)]
            a16 = albuf[pl.ds(g * 16, 16)]
            plsc.addupdate_scatter(suma, [d16], a16)
            plsc.addupdate_scatter(cnt, [d16], ones)
            return 0

        lax.fori_loop(0, SCH // 16, grp, 0)
        return 0

    lax.fori_loop(0, EPW // SCH, chunk, 0)
    pltpu.sync_copy(suma, suma_out.at[wid])
    pltpu.sync_copy(cnt, cnt_out.at[wid])


def _sc_stats_gate(al, dst):
    return pl.kernel(
        _sc_stats_gate_body,
        out_type=[jax.ShapeDtypeStruct((NW, NP), F32),
                  jax.ShapeDtypeStruct((NW, NP), F32)],
        mesh=_MESH,
        compiler_params=pltpu.CompilerParams(needs_layout_passes=False),
        scratch_types=[pltpu.VMEM((SCH,), F32),
                       pltpu.VMEM((SCH,), I32),
                       pltpu.VMEM((NP,), F32),
                       pltpu.VMEM((NP,), F32)],
    )(al, dst)


def _sc_stats_atom_body(ssrc_hbm, sdst_hbm, src_hbm, dst_hbm,
                        al_out, suma_out,
                        srcbuf, dstbuf, albuf, ssrc_v, sdst_v, suma):
    wid = _wid()
    base = wid * EPW
    pltpu.sync_copy(ssrc_hbm, ssrc_v)
    pltpu.sync_copy(sdst_hbm, sdst_v)
    _zero_vec_ref(suma, NP)

    def chunk(c, _):
        off = base + c * SCH
        pltpu.sync_copy(src_hbm.at[pl.ds(off, SCH)], srcbuf)
        pltpu.sync_copy(dst_hbm.at[pl.ds(off, SCH)], dstbuf)

        def grp(g, _):
            s16 = srcbuf[pl.ds(g * 16, 16)]
            d16 = dstbuf[pl.ds(g * 16, 16)]
            a = plsc.load_gather(ssrc_v, [s16]) + plsc.load_gather(sdst_v, [d16])
            a = jnp.maximum(a, 0.01 * a)
            albuf[pl.ds(g * 16, 16)] = a
            plsc.addupdate_scatter(suma, [d16], a)
            return 0

        lax.fori_loop(0, SCH // 16, grp, 0)
        pltpu.sync_copy(albuf, al_out.at[pl.ds(off, SCH)])
        return 0

    lax.fori_loop(0, EPW // SCH, chunk, 0)
    pltpu.sync_copy(suma, suma_out.at[wid])


def _sc_stats_atom(ssrc, sdst, src, dst):
    return pl.kernel(
        _sc_stats_atom_body,
        out_type=[jax.ShapeDtypeStruct((E,), F32),
                  jax.ShapeDtypeStruct((NW, NP), F32)],
        mesh=_MESH,
        compiler_params=pltpu.CompilerParams(needs_layout_passes=False),
        scratch_types=[pltpu.VMEM((SCH,), I32),
                       pltpu.VMEM((SCH,), I32),
                       pltpu.VMEM((SCH,), F32),
                       pltpu.VMEM((NP,), F32),
                       pltpu.VMEM((NP,), F32),
                       pltpu.VMEM((NP,), F32)],
    )(ssrc, sdst, src, dst)


def _sc_pass_b_body(table_hbm, al_hbm, mean_hbm, src_hbm, dst_hbm,
                    numer_out, denom_out,
                    srcv, dstv, albuf, wbuf, rows, zbuf, mean_v, denom,
                    numer_sp, sem):
    cid = lax.axis_index("c")
    sid = lax.axis_index("s")
    wid = cid * NS + sid
    base = wid * EPW
    pltpu.sync_copy(mean_hbm, mean_v)
    _zero_vec_ref(denom, NP)

    def zrow(i, _):
        for j in range(8):
            zbuf[i, pl.ds(j * 16, 16)] = jnp.zeros((16,), F32)
        return 0

    lax.fori_loop(0, 128, zrow, 0)
    for k in range(RPW // 128):
        pltpu.sync_copy(zbuf, numer_sp.at[pl.ds(sid * RPW + k * 128, 128)])
    plsc.subcore_barrier()

    def chunk(c, _):
        off = base + c * GCH
        pltpu.sync_copy(src_hbm.at[pl.ds(off, GCH)], srcv)
        pltpu.sync_copy(dst_hbm.at[pl.ds(off, GCH)], dstv)
        pltpu.sync_copy(al_hbm.at[pl.ds(off, GCH)], albuf)
        pltpu.async_copy(table_hbm.at[srcv], rows, sem).wait()

        def grp(g, _):
            d16 = dstv[pl.ds(g * 16, 16)]
            a16 = albuf[pl.ds(g * 16, 16)]
            w16 = jnp.exp(a16 - plsc.load_gather(mean_v, [d16]))
            wbuf[pl.ds(g * 16, 16)] = w16
            plsc.addupdate_scatter(denom, [d16], w16)
            return 0

        lax.fori_loop(0, GCH // 16, grp, 0)

        def scale(r, _):
            wr = plsc.load_gather(wbuf, [jnp.full((16,), r, I32)])
            for j in range(8):
                rows[r, pl.ds(j * 16, 16)] = rows[r, pl.ds(j * 16, 16)] * wr
            return 0

        lax.fori_loop(0, GCH, scale, 0)
        pltpu.sync_copy(rows, numer_sp.at[dstv], add=True)
        return 0

    lax.fori_loop(0, GNC, chunk, 0)
    plsc.subcore_barrier()
    pltpu.sync_copy(numer_sp.at[pl.ds(sid * RPW, RPW)],
                    numer_out.at[cid, pl.ds(sid * RPW, RPW)])
    pltpu.sync_copy(denom, denom_out.at[wid])


def _sc_pass_b(table, al, mean, src, dst):
    return pl.kernel(
        _sc_pass_b_body,
        out_type=[jax.ShapeDtypeStruct((NC, NP, H), F32),
                  jax.ShapeDtypeStruct((NW, NP), F32)],
        mesh=_MESH,
        compiler_params=pltpu.CompilerParams(needs_layout_passes=False),
        scratch_types=[pltpu.VMEM((GCH,), I32),
                       pltpu.VMEM((GCH,), I32),
                       pltpu.VMEM((GCH,), F32),
                       pltpu.VMEM((GCH,), F32),
                       pltpu.VMEM((GCH, H), F32),
                       pltpu.VMEM((128, H), F32),
                       pltpu.VMEM((NP,), F32),
                       pltpu.VMEM((NP,), F32),
                       pltpu.VMEM_SHARED((NP, H), F32),
                       pltpu.SemaphoreType.DMA],
    )(table, al, mean, src, dst)


NBLK = 10
NROWS = NP // NBLK


def _row_spec(shape_tail):
    return pl.BlockSpec((NROWS,) + shape_tail, lambda i: (i,) + (0,) * len(shape_tail))


def _full_spec(shape):
    return pl.BlockSpec(shape, lambda i: (0,) * len(shape))


def _gru_math(x, h, Wih, Whh, bih, bhh):
    gi = lax.dot_general(x, Wih, (((1,), (1,)), ((), ()))) + bih[None, :]
    gh = lax.dot_general(h, Whh, (((1,), (1,)), ((), ()))) + bhh[None, :]
    ir, iz, inn = jnp.split(gi, 3, axis=-1)
    hr, hz, hn = jnp.split(gh, 3, axis=-1)
    r = jax.nn.sigmoid(ir + hr)
    z = jax.nn.sigmoid(iz + hz)
    nn_ = jnp.tanh(inn + r * hn)
    return (1.0 - z) * nn_ + z * h


def _elu(x):
    return jnp.where(x > 0.0, x, jnp.exp(jnp.minimum(x, 0.0)) - 1.0)


def _lrelu(x):
    return jnp.maximum(x, 0.01 * x)


def _tc_pre_body(x_ref, w1_ref, b1_ref, w1a_ref, gw2_ref, gar_ref,
                 h0_ref, a_ref, g_ref, rdst_ref):
    h0 = _lrelu(lax.dot_general(x_ref[...], w1_ref[...],
                                (((1,), (1,)), ((), ()))) + b1_ref[...])
    h0_ref[...] = h0
    a_ref[...] = lax.dot_general(h0, w1a_ref[...], (((1,), (1,)), ((), ())))
    g_ref[...] = lax.dot_general(h0, gw2_ref[...], (((1,), (1,)), ((), ())))
    rdst_ref[...] = jnp.sum(h0 * gar_ref[...], axis=1, keepdims=True)


def _tc_pre(x_p, w1_p, b1, w1a, gw2, gar):
    return pl.pallas_call(
        _tc_pre_body,
        grid=(NBLK,),
        in_specs=[_row_spec((H,)), _full_spec((H, H)), _full_spec((1, H)),
                  _full_spec((H, H)), _full_spec((H, H)), _full_spec((1, H))],
        out_specs=[_row_spec((H,)), _row_spec((H,)), _row_spec((H,)),
                   _row_spec((1,))],
        out_shape=[jax.ShapeDtypeStruct((NP, H), F32)] * 3
        + [jax.ShapeDtypeStruct((NP, 1), F32)],
    )(x_p, w1_p, b1, w1a, gw2, gar)


EBLK = 8000
ENB = E // EBLK


def _tc_gate_al_body(asrc_ref, ea_ref, rdstg_ref, w1bt_ref, attl_ref, al_ref):
    eb = jnp.dot(ea_ref[...], w1bt_ref[...])
    t = _lrelu(asrc_ref[...] + eb)
    al = jnp.sum(t * attl_ref[...], axis=1, keepdims=True) + rdstg_ref[...]
    al_ref[...] = _lrelu(al)


def _tc_gate_al(asrc, ea, rdstg, w1bt, attl):
    return pl.pallas_call(
        _tc_gate_al_body,
        grid=(ENB,),
        in_specs=[pl.BlockSpec((EBLK, H), lambda i: (i, 0)),
                  pl.BlockSpec((EBLK, ED), lambda i: (i, 0)),
                  pl.BlockSpec((EBLK, 1), lambda i: (i, 0)),
                  _full_spec((ED, H)), _full_spec((1, H))],
        out_specs=pl.BlockSpec((EBLK, 1), lambda i: (i, 0)),
        out_shape=jax.ShapeDtypeStruct((E, 1), F32),
    )(asrc, ea, rdstg, w1bt, attl)


def _tc_mean2_body(suma_ref, cnt_ref, mean_ref, cntt_ref):
    s = jnp.sum(suma_ref[...], axis=0, keepdims=True)
    c = jnp.sum(cnt_ref[...], axis=0, keepdims=True)
    cntt_ref[...] = c
    mean_ref[...] = s / jnp.maximum(c, 1.0)


def _tc_mean2(suma_p, cnt_p):
    return pl.pallas_call(
        _tc_mean2_body,
        out_shape=[jax.ShapeDtypeStruct((1, NP), F32),
                   jax.ShapeDtypeStruct((1, NP), F32)],
    )(suma_p, cnt_p)


def _tc_mean_body(suma_ref, cntt_ref, mean_ref):
    s = jnp.sum(suma_ref[...], axis=0, keepdims=True)
    mean_ref[...] = s / jnp.maximum(cntt_ref[...], 1.0)


def _tc_mean(suma_p, cntt):
    return pl.pallas_call(
        _tc_mean_body,
        out_shape=jax.ShapeDtypeStruct((1, NP), F32),
    )(suma_p, cntt)


def _tc_post_body(mode, numer_ref, denom_ref, bias_ref, hprev_ref,
                  wih_ref, whh_ref, bih_ref, bhh_ref, wn_ref, asn_ref, adn_ref,
                  xh_ref, xs_ref, ssrc_ref, sdst_ref):
    numer = numer_ref[0] + numer_ref[1]
    denom = jnp.sum(denom_ref[...], axis=0)[:, None]
    h = _elu(numer / (denom + 1e-16) + bias_ref[...])
    xh = jnp.maximum(_gru_math(h, hprev_ref[...], wih_ref[...], whh_ref[...],
                               bih_ref[0], bhh_ref[0]), 0.0)
    xh_ref[...] = xh
    xs = lax.dot_general(xh, wn_ref[...], (((1,), (1,)), ((), ())))
    xs_ref[...] = xs
    ssrc_ref[...] = jnp.sum(xs * asn_ref[...], axis=1, keepdims=True)
    if mode == "atom":
        sdst_ref[...] = jnp.sum(xs * adn_ref[...], axis=1, keepdims=True)
    else:
        sdst_ref[...] = ssrc_ref[...]


def _tc_post(mode, numer_p, denom_p, bias, hprev, wih, whh, bih, bhh,
             wn, asn, adn):
    return pl.pallas_call(
        functools.partial(_tc_post_body, mode),
        grid=(NBLK,),
        in_specs=[pl.BlockSpec((NC, NROWS, H), lambda i: (0, i, 0)),
                  pl.BlockSpec((NW, NROWS), lambda i: (0, i)),
                  _full_spec((1, H)), _row_spec((H,)),
                  _full_spec((3 * H, H)), _full_spec((3 * H, H)),
                  _full_spec((1, 3 * H)), _full_spec((1, 3 * H)),
                  _full_spec((H, H)), _full_spec((1, H)), _full_spec((1, H))],
        out_specs=[_row_spec((H,)), _row_spec((H,)),
                   _row_spec((1,)), _row_spec((1,))],
        out_shape=[jax.ShapeDtypeStruct((NP, H), F32),
                   jax.ShapeDtypeStruct((NP, H), F32),
                   jax.ShapeDtypeStruct((NP, 1), F32),
                   jax.ShapeDtypeStruct((NP, 1), F32)],
    )(numer_p, denom_p, bias, hprev, wih, whh, bih, bhh, wn, asn, adn)


def _tc_mol_body(xh_ref, xm_ref, smsrc_ref, batch_ref,
                 molW_ref, mad_ref, mbias_ref,
                 wih_ref, whh_ref, bih_ref, bhh_ref,
                 w2_ref, b2_ref, out_ref):
    xh = xh_ref[...]
    xm = xm_ref[...]
    batch = batch_ref[...]
    gid = lax.broadcasted_iota(I32, (B, 1), 0)
    oht = jnp.where(batch == gid, 1.0, 0.0)
    ones_row = jnp.ones((1, NP), F32)
    counts = lax.dot_general(ones_row, oht, (((1,), (1,)), ((), ())))
    counts = jnp.maximum(counts, 1.0)
    out = jnp.maximum(jnp.dot(oht, xh), 0.0)
    smsrc = smsrc_ref[...]
    for _ in range(T):
        xd = lax.dot_general(out, molW_ref[...], (((1,), (1,)), ((), ())))
        smd = lax.dot_general(mad_ref[...], xd, (((1,), (1,)), ((), ())))
        al = smsrc + jnp.dot(smd, oht)
        al = _lrelu(al)
        suma = lax.dot_general(al, oht, (((1,), (1,)), ((), ())))
        mean = suma / counts
        w = jnp.exp(al - jnp.dot(mean, oht))
        ohtw = oht * w
        denom = jnp.sum(ohtw, axis=1, keepdims=True)
        numer = jnp.dot(ohtw, xm)
        h = _elu(numer / (denom + 1e-16) + mbias_ref[...])
        out = jnp.maximum(_gru_math(h, out, wih_ref[...], whh_ref[...],
                                    bih_ref[0], bhh_ref[0]), 0.0)
    out_ref[...] = lax.dot_general(out, w2_ref[...], (((1,), (1,)), ((), ()))) \
        + b2_ref[...]


def _tc_mol(xh, xm, smsrc, batch_row, molW, mad, mbias,
            wih, whh, bih, bhh, w2, b2):
    return pl.pallas_call(
        _tc_mol_body,
        out_shape=jax.ShapeDtypeStruct((B, OUT), F32),
    )(xh, xm, smsrc, batch_row, molW, mad, mbias, wih, whh, bih, bhh, w2, b2)


def kernel(x, edge_attr, lin1_W, lin1_b, gate_att_l, gate_att_r, gate_W1, gate_W2, gate_bias, gru0_Wih, gru0_Whh, gru0_bih, gru0_bhh, atom_W, atom_att_src, atom_att_dst, atom_bias, agru_Wih, agru_Whh, agru_bih, agru_bhh, mol_W, mol_att_src, mol_att_dst, mol_bias, mgru_Wih, mgru_Whh, mgru_bih, mgru_bhh, lin2_W, lin2_b, edge_index, batch):
    src = edge_index[0]
    dst = edge_index[1]

    x_p = jnp.pad(x, ((0, NP - N), (0, H - IN)))
    w1_p = jnp.pad(lin1_W, ((0, 0), (0, H - IN)))
    w1a = gate_W1[:, :H]
    w1bt = gate_W1[:, H:].T.reshape(ED, H)

    h0, a_tab, g_tab, rdst = _tc_pre(
        x_p, w1_p, lin1_b.reshape(1, H), w1a, gate_W2,
        gate_att_r.reshape(1, H))

    asrc, rdstg = _sc_gate_gather(a_tab, rdst.reshape(NP), src, dst)
    al = _tc_gate_al(asrc, edge_attr, rdstg.reshape(E, 1), w1bt,
                     gate_att_l.reshape(1, H))
    al = al.reshape(E)
    suma_p, cnt_p = _sc_stats_gate(al, dst)
    mean, cntt = _tc_mean2(suma_p, cnt_p)
    numer_p, denom_p = _sc_pass_b(g_tab, al, mean.reshape(NP), src, dst)

    xh, xs, ssrc, sdst = _tc_post(
        "atom", numer_p, denom_p, gate_bias.reshape(1, H), h0,
        gru0_Wih, gru0_Whh, gru0_bih.reshape(1, 3 * H),
        gru0_bhh.reshape(1, 3 * H),
        atom_W[0], atom_att_src[0].reshape(1, H), atom_att_dst[0].reshape(1, H))

    for l in range(L - 1):
        al_l, suma_p = _sc_stats_atom(ssrc.reshape(NP), sdst.reshape(NP), src, dst)
        mean = _tc_mean(suma_p, cntt)
        numer_p, denom_p = _sc_pass_b(xs, al_l, mean.reshape(NP), src, dst)
        last = l == L - 2
        if last:
            wn, asn, adn = mol_W, mol_att_src, mol_att_src
        else:
            wn, asn, adn = atom_W[l + 1], atom_att_src[l + 1], atom_att_dst[l + 1]
        xh, xs, ssrc, sdst = _tc_post(
            "mol" if last else "atom",
            numer_p, denom_p, atom_bias[l].reshape(1, H), xh,
            agru_Wih[l], agru_Whh[l], agru_bih[l].reshape(1, 3 * H),
            agru_bhh[l].reshape(1, 3 * H),
            wn, asn.reshape(1, H), adn.reshape(1, H))

    return _tc_mol(
        xh, xs, ssrc.reshape(1, NP),
        jnp.pad(batch, (0, NP - N), constant_values=B).reshape(1, NP),
        mol_W, mol_att_dst.reshape(1, H), mol_bias.reshape(1, H),
        mgru_Wih, mgru_Whh, mgru_bih.reshape(1, 3 * H),
        mgru_bhh.reshape(1, 3 * H), lin2_W, lin2_b.reshape(1, OUT))


# speedup vs baseline: 22.6769x; 1.6610x over previous
import functools

import jax
import jax.numpy as jnp
from jax import lax
from jax.experimental import pallas as pl
from jax.experimental.pallas import tpu as pltpu
from jax.experimental.pallas import tpu_sc as plsc

N = 10000
E = 320000
IN = 9
H = 128
OUT = 128
ED = 3
L = 4
T = 2
B = 512

NP = 10240
NC = 2
NS = 16
NW = NC * NS
EPW = E // NW
RPW = NP // NS

_MESH = plsc.VectorSubcoreMesh(core_axis_name="c", subcore_axis_name="s")

F32 = jnp.float32
I32 = jnp.int32


def _wid():
    return lax.axis_index("c") * NS + lax.axis_index("s")


def _zero_vec_ref(ref, n):
    z = jnp.zeros((16,), F32)

    def body(i, _):
        ref[pl.ds(i * 16, 16)] = z
        return 0

    lax.fori_loop(0, n // 16, body, 0)


GCH = 80
GNC = EPW // GCH
NCHT = EPW // GCH
NCB = 5
NCJ = NCHT // NCB
ER = E // GCH


def _sc_gate_gather_body(a_hbm, rdst_hbm, src_hbm, dst_hbm,
                         asrc_out, rdstg_out,
                         srcb, dstb, rows0, rows1, rdbuf, rdst_v, sem0, sem1):
    wid = _wid()
    brow = wid * NCHT
    pltpu.sync_copy(rdst_hbm, rdst_v)
    pltpu.sync_copy(src_hbm.at[wid], srcb)
    pltpu.sync_copy(dst_hbm.at[wid], dstb)

    @pl.loop(0, NCHT)
    def rg(j):
        for g in range(GCH // 16):
            d16 = dstb[j, pl.ds(g * 16, 16)]
            rdbuf[j, pl.ds(g * 16, 16)] = plsc.load_gather(rdst_v, [d16])

    pltpu.sync_copy(rdbuf, rdstg_out.at[wid])

    def issue(c, rows, sem):
        pltpu.async_copy(a_hbm.at[srcb.at[c]], rows, sem)

    def process(c, rows, sem):
        pltpu.make_async_copy(a_hbm.at[srcb.at[c]], rows, sem).wait()
        pltpu.sync_copy(rows, asrc_out.at[pl.ds((brow + c) * GCH, GCH)])

    issue(0, rows0, sem0)

    @pl.loop(0, NCHT - 1, step=2)
    def chunk(c):
        issue(c + 1, rows1, sem1)
        process(c, rows0, sem0)

        @pl.when(c + 2 < NCHT)
        def _():
            issue(c + 2, rows0, sem0)

        process(c + 1, rows1, sem1)

    process(NCHT - 1, rows0, sem0)


def _sc_gate_gather(a, rdst, src2, dst2):
    return pl.kernel(
        _sc_gate_gather_body,
        out_type=[jax.ShapeDtypeStruct((E, H), F32),
                  jax.ShapeDtypeStruct((NW, NCHT, GCH), F32)],
        mesh=_MESH,
        compiler_params=pltpu.CompilerParams(needs_layout_passes=False),
        scratch_types=[pltpu.VMEM((NCHT, GCH), I32),
                       pltpu.VMEM((NCHT, GCH), I32),
                       pltpu.VMEM((GCH, H), F32),
                       pltpu.VMEM((GCH, H), F32),
                       pltpu.VMEM((NCHT, GCH), F32),
                       pltpu.VMEM((NP,), F32),
                       pltpu.SemaphoreType.DMA,
                       pltpu.SemaphoreType.DMA],
    )(a, rdst, src2, dst2)


SCH = 10000


def _sc_stats_gate_body(al_hbm, dst_hbm, suma_out, cnt_out,
                        albuf, dstbuf, suma, cnt):
    wid = _wid()
    base = wid * EPW
    _zero_vec_ref(suma, NP)
    _zero_vec_ref(cnt, NP)
    ones = jnp.full((16,), 1.0, F32)

    def chunk(c, _):
        off = base + c * SCH
        pltpu.sync_copy(al_hbm.at[pl.ds(off, SCH)], albuf)
        pltpu.sync_copy(dst_hbm.at[pl.ds(off, SCH)], dstbuf)

        def grp(g, _):
            d16 = dstbuf[pl.ds(g * 16, 16)]
            a16 = albuf[pl.ds(g * 16, 16)]
            plsc.addupdate_scatter(suma, [d16], a16)
            plsc.addupdate_scatter(cnt, [d16], ones)
            return 0

        lax.fori_loop(0, SCH // 16, grp, 0)
        return 0

    lax.fori_loop(0, EPW // SCH, chunk, 0)
    pltpu.sync_copy(suma, suma_out.at[wid])
    pltpu.sync_copy(cnt, cnt_out.at[wid])


def _sc_stats_gate(al, dst):
    return pl.kernel(
        _sc_stats_gate_body,
        out_type=[jax.ShapeDtypeStruct((NW, NP), F32),
                  jax.ShapeDtypeStruct((NW, NP), F32)],
        mesh=_MESH,
        compiler_params=pltpu.CompilerParams(needs_layout_passes=False),
        scratch_types=[pltpu.VMEM((SCH,), F32),
                       pltpu.VMEM((SCH,), I32),
                       pltpu.VMEM((NP,), F32),
                       pltpu.VMEM((NP,), F32)],
    )(al, dst)


def _sc_stats_atom_body(ssrc_hbm, sdst_hbm, src_hbm, dst_hbm,
                        al_out, suma_out,
                        srcbuf, dstbuf, albuf, ssrc_v, sdst_v, suma):
    wid = _wid()
    base = wid * EPW
    pltpu.sync_copy(ssrc_hbm, ssrc_v)
    pltpu.sync_copy(sdst_hbm, sdst_v)
    _zero_vec_ref(suma, NP)

    def chunk(c, _):
        off = base + c * SCH
        pltpu.sync_copy(src_hbm.at[pl.ds(off, SCH)], srcbuf)
        pltpu.sync_copy(dst_hbm.at[pl.ds(off, SCH)], dstbuf)

        def grp(g, _):
            s16 = srcbuf[pl.ds(g * 16, 16)]
            d16 = dstbuf[pl.ds(g * 16, 16)]
            a = plsc.load_gather(ssrc_v, [s16]) + plsc.load_gather(sdst_v, [d16])
            a = jnp.maximum(a, 0.01 * a)
            albuf[pl.ds(g * 16, 16)] = a
            plsc.addupdate_scatter(suma, [d16], a)
            return 0

        lax.fori_loop(0, SCH // 16, grp, 0)
        pltpu.sync_copy(albuf, al_out.at[pl.ds(off, SCH)])
        return 0

    lax.fori_loop(0, EPW // SCH, chunk, 0)
    pltpu.sync_copy(suma, suma_out.at[wid])


def _sc_stats_atom(ssrc, sdst, src, dst):
    return pl.kernel(
        _sc_stats_atom_body,
        out_type=[jax.ShapeDtypeStruct((E,), F32),
                  jax.ShapeDtypeStruct((NW, NP), F32)],
        mesh=_MESH,
        compiler_params=pltpu.CompilerParams(needs_layout_passes=False),
        scratch_types=[pltpu.VMEM((SCH,), I32),
                       pltpu.VMEM((SCH,), I32),
                       pltpu.VMEM((SCH,), F32),
                       pltpu.VMEM((NP,), F32),
                       pltpu.VMEM((NP,), F32),
                       pltpu.VMEM((NP,), F32)],
    )(ssrc, sdst, src, dst)


def _sc_w_body(al_hbm, mean_hbm, dst_hbm, w_out, denom_out,
               ab, db, mean_v, denom):
    wid = _wid()
    pltpu.sync_copy(mean_hbm, mean_v)
    _zero_vec_ref(denom, NP)
    for b in range(NCB):
        pltpu.sync_copy(al_hbm.at[wid, b], ab)
        pltpu.sync_copy(dst_hbm.at[wid, b], db)

        @pl.loop(0, NCJ)
        def wloop(j):
            for g in range(GCH // 16):
                d16 = db[j, pl.ds(g * 16, 16)]
                a16 = ab[j, pl.ds(g * 16, 16)]
                w16 = jnp.exp(a16 - plsc.load_gather(mean_v, [d16]))
                ab[j, pl.ds(g * 16, 16)] = w16
                plsc.addupdate_scatter(denom, [d16], w16)

        pltpu.sync_copy(ab, w_out.at[wid, b])
    pltpu.sync_copy(denom, denom_out.at[wid])


def _sc_w(al4, dst4, mean):
    return pl.kernel(
        _sc_w_body,
        out_type=[jax.ShapeDtypeStruct((NW, NCB, NCJ, GCH), F32),
                  jax.ShapeDtypeStruct((NW, NP), F32)],
        mesh=_MESH,
        compiler_params=pltpu.CompilerParams(needs_layout_passes=False),
        scratch_types=[pltpu.VMEM((NCJ, GCH), F32),
                       pltpu.VMEM((NCJ, GCH), I32),
                       pltpu.VMEM((NP,), F32),
                       pltpu.VMEM((NP,), F32)],
    )(al4, mean, dst4)


def _sc_pass_b_body(table_hbm, w_hbm, src_hbm, dst_hbm,
                    numer_out,
                    srcb, dstb, wb, rows0, rows1, numer_sp, sem0, sem1):
    cid = lax.axis_index("c")
    sid = lax.axis_index("s")
    wid = cid * NS + sid

    @pl.loop(0, GCH)
    def zrow(i):
        for j in range(8):
            rows0[i, pl.ds(j * 16, 16)] = jnp.zeros((16,), F32)

    for k in range(RPW // GCH):
        pltpu.sync_copy(rows0, numer_sp.at[pl.ds(sid * RPW + k * GCH, GCH)])
    plsc.subcore_barrier()

    def issue(b, j, rows, sem):
        pltpu.async_copy(table_hbm.at[srcb.at[j]], rows, sem)

    def process(b, j, rows, sem):
        pltpu.make_async_copy(table_hbm.at[srcb.at[j]], rows, sem).wait()

        @pl.loop(0, GCH, unroll=2)
        def scale(r):
            wr = plsc.load_gather(wb, [jnp.full((16,), j, I32),
                                       jnp.full((16,), r, I32)])
            for q in range(8):
                rows[r, pl.ds(q * 16, 16)] = rows[r, pl.ds(q * 16, 16)] * wr

        pltpu.sync_copy(rows, numer_sp.at[dstb.at[j]], add=True)

    for b in range(NCB):
        pltpu.sync_copy(src_hbm.at[wid, b], srcb)
        pltpu.sync_copy(dst_hbm.at[wid, b], dstb)
        pltpu.sync_copy(w_hbm.at[wid, b], wb)
        issue(b, 0, rows0, sem0)

        @pl.loop(0, NCJ - 1, step=2)
        def chunk(j):
            issue(b, j + 1, rows1, sem1)
            process(b, j, rows0, sem0)

            @pl.when(j + 2 < NCJ)
            def _():
                issue(b, j + 2, rows0, sem0)

            process(b, j + 1, rows1, sem1)

        process(b, NCJ - 1, rows0, sem0)

    plsc.subcore_barrier()
    pltpu.sync_copy(numer_sp.at[pl.ds(sid * RPW, RPW)],
                    numer_out.at[cid, pl.ds(sid * RPW, RPW)])


def _sc_pass_b(table, w4, src4, dst4):
    return pl.kernel(
        _sc_pass_b_body,
        out_type=jax.ShapeDtypeStruct((NC, NP, H), F32),
        mesh=_MESH,
        compiler_params=pltpu.CompilerParams(needs_layout_passes=False),
        scratch_types=[pltpu.VMEM((NCJ, GCH), I32),
                       pltpu.VMEM((NCJ, GCH), I32),
                       pltpu.VMEM((NCJ, GCH), F32),
                       pltpu.VMEM((GCH, H), F32),
                       pltpu.VMEM((GCH, H), F32),
                       pltpu.VMEM_SHARED((NP, H), F32),
                       pltpu.SemaphoreType.DMA,
                       pltpu.SemaphoreType.DMA],
    )(table, w4, src4, dst4)


NBLK = 10
NROWS = NP // NBLK


def _row_spec(shape_tail):
    return pl.BlockSpec((NROWS,) + shape_tail, lambda i: (i,) + (0,) * len(shape_tail))


def _full_spec(shape):
    return pl.BlockSpec(shape, lambda i: (0,) * len(shape))


def _gru_math(x, h, Wih, Whh, bih, bhh):
    gi = lax.dot_general(x, Wih, (((1,), (1,)), ((), ()))) + bih[None, :]
    gh = lax.dot_general(h, Whh, (((1,), (1,)), ((), ()))) + bhh[None, :]
    ir, iz, inn = jnp.split(gi, 3, axis=-1)
    hr, hz, hn = jnp.split(gh, 3, axis=-1)
    r = jax.nn.sigmoid(ir + hr)
    z = jax.nn.sigmoid(iz + hz)
    nn_ = jnp.tanh(inn + r * hn)
    return (1.0 - z) * nn_ + z * h


def _elu(x):
    return jnp.where(x > 0.0, x, jnp.exp(jnp.minimum(x, 0.0)) - 1.0)


def _lrelu(x):
    return jnp.maximum(x, 0.01 * x)


def _tc_pre_body(x_ref, w1_ref, b1_ref, w1a_ref, gw2_ref, gar_ref,
                 h0_ref, a_ref, g_ref, rdst_ref):
    h0 = _lrelu(lax.dot_general(x_ref[...], w1_ref[...],
                                (((1,), (1,)), ((), ()))) + b1_ref[...])
    h0_ref[...] = h0
    a_ref[...] = lax.dot_general(h0, w1a_ref[...], (((1,), (1,)), ((), ())))
    g_ref[...] = lax.dot_general(h0, gw2_ref[...], (((1,), (1,)), ((), ())))
    rdst_ref[...] = jnp.sum(h0 * gar_ref[...], axis=1, keepdims=True)


def _tc_pre(x_p, w1_p, b1, w1a, gw2, gar):
    return pl.pallas_call(
        _tc_pre_body,
        grid=(NBLK,),
        in_specs=[_row_spec((H,)), _full_spec((H, H)), _full_spec((1, H)),
                  _full_spec((H, H)), _full_spec((H, H)), _full_spec((1, H))],
        out_specs=[_row_spec((H,)), _row_spec((H,)), _row_spec((H,)),
                   _row_spec((1,))],
        out_shape=[jax.ShapeDtypeStruct((NP, H), F32)] * 3
        + [jax.ShapeDtypeStruct((NP, 1), F32)],
    )(x_p, w1_p, b1, w1a, gw2, gar)


EBLK = 8000
ENB = E // EBLK


def _tc_gate_al_body(asrc_ref, ea_ref, rdstg_ref, w1bt_ref, attl_ref, al_ref):
    eb = jnp.dot(ea_ref[...], w1bt_ref[...])
    t = _lrelu(asrc_ref[...] + eb)
    al = jnp.sum(t * attl_ref[...], axis=1, keepdims=True) + rdstg_ref[...]
    al_ref[...] = _lrelu(al)


def _tc_gate_al(asrc, ea, rdstg, w1bt, attl):
    return pl.pallas_call(
        _tc_gate_al_body,
        grid=(ENB,),
        in_specs=[pl.BlockSpec((EBLK, H), lambda i: (i, 0)),
                  pl.BlockSpec((EBLK, ED), lambda i: (i, 0)),
                  pl.BlockSpec((EBLK, 1), lambda i: (i, 0)),
                  _full_spec((ED, H)), _full_spec((1, H))],
        out_specs=pl.BlockSpec((EBLK, 1), lambda i: (i, 0)),
        out_shape=jax.ShapeDtypeStruct((E, 1), F32),
    )(asrc, ea, rdstg, w1bt, attl)


def _tc_mean2_body(suma_ref, cnt_ref, mean_ref, cntt_ref):
    s = jnp.sum(suma_ref[...], axis=0, keepdims=True)
    c = jnp.sum(cnt_ref[...], axis=0, keepdims=True)
    cntt_ref[...] = c
    mean_ref[...] = s / jnp.maximum(c, 1.0)


def _tc_mean2(suma_p, cnt_p):
    return pl.pallas_call(
        _tc_mean2_body,
        out_shape=[jax.ShapeDtypeStruct((1, NP), F32),
                   jax.ShapeDtypeStruct((1, NP), F32)],
    )(suma_p, cnt_p)


def _tc_mean_body(suma_ref, cntt_ref, mean_ref):
    s = jnp.sum(suma_ref[...], axis=0, keepdims=True)
    mean_ref[...] = s / jnp.maximum(cntt_ref[...], 1.0)


def _tc_mean(suma_p, cntt):
    return pl.pallas_call(
        _tc_mean_body,
        out_shape=jax.ShapeDtypeStruct((1, NP), F32),
    )(suma_p, cntt)


def _tc_post_body(mode, numer_ref, denom_ref, bias_ref, hprev_ref,
                  wih_ref, whh_ref, bih_ref, bhh_ref, wn_ref, asn_ref, adn_ref,
                  xh_ref, xs_ref, ssrc_ref, sdst_ref):
    numer = numer_ref[0] + numer_ref[1]
    denom = jnp.sum(denom_ref[...], axis=0)[:, None]
    h = _elu(numer / (denom + 1e-16) + bias_ref[...])
    xh = jnp.maximum(_gru_math(h, hprev_ref[...], wih_ref[...], whh_ref[...],
                               bih_ref[0], bhh_ref[0]), 0.0)
    xh_ref[...] = xh
    xs = lax.dot_general(xh, wn_ref[...], (((1,), (1,)), ((), ())))
    xs_ref[...] = xs
    ssrc_ref[...] = jnp.sum(xs * asn_ref[...], axis=1, keepdims=True)
    if mode == "atom":
        sdst_ref[...] = jnp.sum(xs * adn_ref[...], axis=1, keepdims=True)
    else:
        sdst_ref[...] = ssrc_ref[...]


def _tc_post(mode, numer_p, denom_p, bias, hprev, wih, whh, bih, bhh,
             wn, asn, adn):
    return pl.pallas_call(
        functools.partial(_tc_post_body, mode),
        grid=(NBLK,),
        in_specs=[pl.BlockSpec((NC, NROWS, H), lambda i: (0, i, 0)),
                  pl.BlockSpec((NW, NROWS), lambda i: (0, i)),
                  _full_spec((1, H)), _row_spec((H,)),
                  _full_spec((3 * H, H)), _full_spec((3 * H, H)),
                  _full_spec((1, 3 * H)), _full_spec((1, 3 * H)),
                  _full_spec((H, H)), _full_spec((1, H)), _full_spec((1, H))],
        out_specs=[_row_spec((H,)), _row_spec((H,)),
                   _row_spec((1,)), _row_spec((1,))],
        out_shape=[jax.ShapeDtypeStruct((NP, H), F32),
                   jax.ShapeDtypeStruct((NP, H), F32),
                   jax.ShapeDtypeStruct((NP, 1), F32),
                   jax.ShapeDtypeStruct((NP, 1), F32)],
    )(numer_p, denom_p, bias, hprev, wih, whh, bih, bhh, wn, asn, adn)


def _tc_mol_body(xh_ref, xm_ref, smsrc_ref, batch_ref,
                 molW_ref, mad_ref, mbias_ref,
                 wih_ref, whh_ref, bih_ref, bhh_ref,
                 w2_ref, b2_ref, out_ref):
    xh = xh_ref[...]
    xm = xm_ref[...]
    batch = batch_ref[...]
    gid = lax.broadcasted_iota(I32, (B, 1), 0)
    oht = jnp.where(batch == gid, 1.0, 0.0)
    ones_row = jnp.ones((1, NP), F32)
    counts = lax.dot_general(ones_row, oht, (((1,), (1,)), ((), ())))
    counts = jnp.maximum(counts, 1.0)
    out = jnp.maximum(jnp.dot(oht, xh), 0.0)
    smsrc = smsrc_ref[...]
    for _ in range(T):
        xd = lax.dot_general(out, molW_ref[...], (((1,), (1,)), ((), ())))
        smd = lax.dot_general(mad_ref[...], xd, (((1,), (1,)), ((), ())))
        al = smsrc + jnp.dot(smd, oht)
        al = _lrelu(al)
        suma = lax.dot_general(al, oht, (((1,), (1,)), ((), ())))
        mean = suma / counts
        w = jnp.exp(al - jnp.dot(mean, oht))
        ohtw = oht * w
        denom = jnp.sum(ohtw, axis=1, keepdims=True)
        numer = jnp.dot(ohtw, xm)
        h = _elu(numer / (denom + 1e-16) + mbias_ref[...])
        out = jnp.maximum(_gru_math(h, out, wih_ref[...], whh_ref[...],
                                    bih_ref[0], bhh_ref[0]), 0.0)
    out_ref[...] = lax.dot_general(out, w2_ref[...], (((1,), (1,)), ((), ()))) \
        + b2_ref[...]


def _tc_mol(xh, xm, smsrc, batch_row, molW, mad, mbias,
            wih, whh, bih, bhh, w2, b2):
    return pl.pallas_call(
        _tc_mol_body,
        out_shape=jax.ShapeDtypeStruct((B, OUT), F32),
    )(xh, xm, smsrc, batch_row, molW, mad, mbias, wih, whh, bih, bhh, w2, b2)


def kernel(x, edge_attr, lin1_W, lin1_b, gate_att_l, gate_att_r, gate_W1, gate_W2, gate_bias, gru0_Wih, gru0_Whh, gru0_bih, gru0_bhh, atom_W, atom_att_src, atom_att_dst, atom_bias, agru_Wih, agru_Whh, agru_bih, agru_bhh, mol_W, mol_att_src, mol_att_dst, mol_bias, mgru_Wih, mgru_Whh, mgru_bih, mgru_bhh, lin2_W, lin2_b, edge_index, batch):
    src = edge_index[0]
    dst = edge_index[1]
    src2 = src.reshape(NW, NCHT, GCH)
    dst2 = dst.reshape(NW, NCHT, GCH)
    src4 = src.reshape(NW, NCB, NCJ, GCH)
    dst4 = dst.reshape(NW, NCB, NCJ, GCH)

    x_p = jnp.pad(x, ((0, NP - N), (0, H - IN)))
    w1_p = jnp.pad(lin1_W, ((0, 0), (0, H - IN)))
    w1a = gate_W1[:, :H]
    w1bt = gate_W1[:, H:].T.reshape(ED, H)

    h0, a_tab, g_tab, rdst = _tc_pre(
        x_p, w1_p, lin1_b.reshape(1, H), w1a, gate_W2,
        gate_att_r.reshape(1, H))

    asrc, rdstg = _sc_gate_gather(a_tab, rdst.reshape(NP), src2, dst2)
    al = _tc_gate_al(asrc, edge_attr, rdstg.reshape(E, 1), w1bt,
                     gate_att_l.reshape(1, H))
    al = al.reshape(E)
    suma_p, cnt_p = _sc_stats_gate(al, dst)
    mean, cntt = _tc_mean2(suma_p, cnt_p)
    w4, denom_p = _sc_w(al.reshape(NW, NCB, NCJ, GCH), dst4, mean.reshape(NP))
    numer_p = _sc_pass_b(g_tab, w4, src4, dst4)

    xh, xs, ssrc, sdst = _tc_post(
        "atom", numer_p, denom_p, gate_bias.reshape(1, H), h0,
        gru0_Wih, gru0_Whh, gru0_bih.reshape(1, 3 * H),
        gru0_bhh.reshape(1, 3 * H),
        atom_W[0], atom_att_src[0].reshape(1, H), atom_att_dst[0].reshape(1, H))

    for l in range(L - 1):
        al_l, suma_p = _sc_stats_atom(ssrc.reshape(NP), sdst.reshape(NP), src, dst)
        mean = _tc_mean(suma_p, cntt)
        w4, denom_p = _sc_w(al_l.reshape(NW, NCB, NCJ, GCH), dst4, mean.reshape(NP))
        numer_p = _sc_pass_b(xs, w4, src4, dst4)
        last = l == L - 2
        if last:
            wn, asn, adn = mol_W, mol_att_src, mol_att_src
        else:
            wn, asn, adn = atom_W[l + 1], atom_att_src[l + 1], atom_att_dst[l + 1]
        xh, xs, ssrc, sdst = _tc_post(
            "mol" if last else "atom",
            numer_p, denom_p, atom_bias[l].reshape(1, H), xh,
            agru_Wih[l], agru_Whh[l], agru_bih[l].reshape(1, 3 * H),
            agru_bhh[l].reshape(1, 3 * H),
            wn, asn.reshape(1, H), adn.reshape(1, H))

    return _tc_mol(
        xh, xs, ssrc.reshape(1, NP),
        jnp.pad(batch, (0, NP - N), constant_values=B).reshape(1, NP),
        mol_W, mol_att_dst.reshape(1, H), mol_bias.reshape(1, H),
        mgru_Wih, mgru_Whh, mgru_bih.reshape(1, 3 * H),
        mgru_bhh.reshape(1, 3 * H), lin2_W, lin2_b.reshape(1, OUT))
